# Initial kernel scaffold; baseline (speedup 1.0000x reference)
import jax
import jax.numpy as jnp
from jax.experimental import pallas as pl


def kernel(x, edge_index, W1, a_s1, a_d1, b1, W2, a_s2, a_d2, b2):
    raise NotImplementedError("write your pallas kernel here")


# speedup vs baseline: 28.5849x; 28.5849x over previous
import functools

import jax
import jax.numpy as jnp
from jax import lax
from jax.experimental import pallas as pl
from jax.experimental.pallas import tpu as pltpu
from jax.experimental.pallas import tpu_sc as plsc

N = 10000
NROW = 10240
E = 320000
NW = 32
EPW = E // NW
CHUNK = 80
NCHUNK = EPW // CHUNK
ROWS_PER_TILE = NROW // 16
NPAD = 10112


def _leaky(v):
    return jnp.where(v > 0, v, 0.2 * v)


def _tc_pre_body(x_ref, w_ref, as_ref, ad_ref,
                 table_ref, init_ref, asv_ref, adv_ref, asmax_ref):
    h = jnp.dot(x_ref[:], w_ref[:], preferred_element_type=jnp.float32)
    asv = (h * as_ref[:]).sum(-1, keepdims=True)
    adv = (h * ad_ref[:]).sum(-1, keepdims=True)
    asmax = jnp.max(asv)
    s_self = asv + adv
    cd_self = _leaky(asmax + adv)
    wself = jnp.exp(_leaky(s_self) - cd_self)
    hid = h.shape[1]
    width = table_ref.shape[1]
    ones = jnp.ones((h.shape[0], 1), jnp.float32)
    pad = jnp.zeros((h.shape[0], width - hid - 1), jnp.float32)
    table = jnp.concatenate([h, ones, pad], axis=1)
    rowpad = jnp.zeros((NROW - h.shape[0], width), jnp.float32)
    table_ref[:] = jnp.concatenate([table, rowpad], axis=0)
    init_ref[0] = jnp.concatenate([wself * table, rowpad], axis=0)
    init_ref[1] = jnp.zeros((NROW, width), jnp.float32)
    asv_ref[:] = asv
    adv_ref[:] = adv
    asmax_ref[:] = jnp.full((1, 1), asmax, jnp.float32)


def _tc_pre(h_in, W, a_s, a_d, width):
    n = h_in.shape[0]
    out_shape = (
        jax.ShapeDtypeStruct((NROW, width), jnp.float32),
        jax.ShapeDtypeStruct((2, NROW, width), jnp.float32),
        jax.ShapeDtypeStruct((n, 1), jnp.float32),
        jax.ShapeDtypeStruct((n, 1), jnp.float32),
        jax.ShapeDtypeStruct((1, 1), jnp.float32),
    )
    return pl.pallas_call(_tc_pre_body, out_shape=out_shape)(
        h_in, W, a_s.reshape(1, -1), a_d.reshape(1, -1))


def _tc_mid_body(acc_ref, b_ref, hid_s_ref, out_ref):
    hid = hid_s_ref.shape[1]
    n = out_ref.shape[0]
    a = acc_ref[0, :n] + acc_ref[1, :n]
    num = a[:, :hid]
    den = a[:, hid:hid + 1]
    out_ref[:] = jax.nn.relu(num / den + b_ref[:])


def _tc_mid(acc, b, hid):
    return pl.pallas_call(
        _tc_mid_body,
        out_shape=jax.ShapeDtypeStruct((N, hid), jnp.float32),
    )(acc, b.reshape(1, -1), jnp.zeros((1, hid), jnp.float32))


def _tc_post_body(acc_ref, b_ref, hid_s_ref, out_ref):
    hid = hid_s_ref.shape[1]
    n = out_ref.shape[0]
    a = acc_ref[0, :n] + acc_ref[1, :n]
    num = a[:, :hid]
    den = a[:, hid:hid + 1]
    out_ref[:] = num / den + b_ref[:]


def _tc_post(acc, b, hid):
    return pl.pallas_call(
        _tc_post_body,
        out_shape=jax.ShapeDtypeStruct((N, hid), jnp.float32),
    )(acc, b.reshape(1, -1), jnp.zeros((1, hid), jnp.float32))


def _make_sc(width):
    nb = width // 16
    mesh = plsc.VectorSubcoreMesh(core_axis_name="c", subcore_axis_name="s")

    @functools.partial(
        pl.kernel,
        out_type=jax.ShapeDtypeStruct((2, NROW, width), jnp.float32),
        mesh=mesh,
        compiler_params=pltpu.CompilerParams(
            needs_layout_passes=False, use_tc_tiling_on_sc=False),
        scratch_types=[
            pltpu.VMEM((NPAD,), jnp.float32),
            pltpu.VMEM((NPAD,), jnp.float32),
            pltpu.VMEM((CHUNK,), jnp.int32),
            pltpu.VMEM((CHUNK,), jnp.int32),
            pltpu.VMEM((CHUNK, width), jnp.float32),
            pltpu.VMEM((CHUNK,), jnp.float32),
            pltpu.VMEM_SHARED((NROW, width), jnp.float32),
            pltpu.SemaphoreType.DMA,
        ],
    )
    def sc_edge_pass(table_hbm, init_hbm, asv_hbm, adv_hbm, esrc_hbm,
                     edst_hbm, out_hbm, as_v, ad_v, src_v, dst_v, rows_v,
                     w_v, acc_sh, sem):
        c = lax.axis_index("c")
        s = lax.axis_index("s")
        wid = c * 16 + s
        r0 = s * ROWS_PER_TILE
        pltpu.sync_copy(init_hbm.at[c, pl.ds(r0, ROWS_PER_TILE)],
                        acc_sh.at[pl.ds(r0, ROWS_PER_TILE)])
        pltpu.sync_copy(asv_hbm, as_v)
        pltpu.sync_copy(adv_hbm, ad_v)
        plsc.subcore_barrier()

        base_e = wid * EPW

        def chunk_body(j, carry):
            e0 = base_e + j * CHUNK
            pltpu.sync_copy(esrc_hbm.at[pl.ds(e0, CHUNK)], src_v)
            pltpu.sync_copy(edst_hbm.at[pl.ds(e0, CHUNK)], dst_v)
            pltpu.async_copy(table_hbm.at[src_v], rows_v, sem).wait()
            cvec = as_v[pl.ds(N, 16)]
            for g in range(CHUNK // 16):
                sidx = src_v[pl.ds(g * 16, 16)]
                didx = dst_v[pl.ds(g * 16, 16)]
                sv = plsc.load_gather(as_v, [sidx])
                dv = plsc.load_gather(ad_v, [didx])
                w = jnp.exp(_leaky(sv + dv) - _leaky(cvec + dv))
                w_v[pl.ds(g * 16, 16)] = w

            def row_body(i, carry2):
                wi = plsc.load_gather(w_v, [jnp.full((16,), i, jnp.int32)])
                for r in range(nb):
                    rows_v[i, pl.ds(r * 16, 16)] = (
                        rows_v[i, pl.ds(r * 16, 16)] * wi)
                return carry2

            lax.fori_loop(0, CHUNK, row_body, 0)
            pltpu.sync_copy(rows_v, acc_sh.at[dst_v], add=True)
            return carry

        lax.fori_loop(0, NCHUNK, chunk_body, 0)
        plsc.subcore_barrier()
        pltpu.sync_copy(acc_sh.at[pl.ds(r0, ROWS_PER_TILE)],
                        out_hbm.at[c, pl.ds(r0, ROWS_PER_TILE)])

    return sc_edge_pass


_sc_pass_32 = _make_sc(32)
_sc_pass_48 = _make_sc(48)


def _pad_scores(asv, adv, asmax):
    a_s = jnp.concatenate(
        [asv[:, 0], jnp.full((16,), asmax[0, 0], jnp.float32),
         jnp.zeros((NPAD - N - 16,), jnp.float32)])
    a_d = jnp.concatenate([adv[:, 0], jnp.zeros((NPAD - N,), jnp.float32)])
    return a_s, a_d


def _gat_layer(h_in, edges, W, a_s, a_d, b, width, sc_pass, final):
    table, init, asv, adv, asmax = _tc_pre(h_in, W, a_s, a_d, width)
    asv_p, adv_p = _pad_scores(asv, adv, asmax)
    acc = sc_pass(table, init, asv_p, adv_p, edges[0], edges[1])
    hid = W.shape[1]
    if final:
        return _tc_post(acc, b, hid)
    return _tc_mid(acc, b, hid)


def kernel(x, edge_index, W1, a_s1, a_d1, b1, W2, a_s2, a_d2, b2):
    edges = edge_index.astype(jnp.int32)
    h1 = _gat_layer(x, edges, W1, a_s1, a_d1, b1, 32, _sc_pass_32,
                    final=False)
    out = _gat_layer(h1, edges, W2, a_s2, a_d2, b2, 48, _sc_pass_48,
                     final=True)
    return out
